# baseline (speedup 1.0000x reference)
import functools

import jax
import jax.numpy as jnp
from jax import lax
from jax.experimental import pallas as pl
from jax.experimental.pallas import tpu as pltpu
from jax.experimental.pallas import tpu_sc as plsc

VOCAB = 1000000
EMBED = 64
HIDDEN = 256
CLASSES = 5
BATCH = 4096
SEQ = 200

_INFO = plsc.get_sparse_core_info()
NC = _INFO.num_cores
NS = _INFO.num_subcores
LANES = _INFO.num_lanes
NW = NC * NS
BPW = BATCH // NW
C0 = 104
C1 = SEQ - C0
EG = EMBED // LANES
HALF = SEQ // 2


def _pool_body(x_hbm, table_hbm, out_hbm, idx_v, rows_v, out_v, sem):
    wid = lax.axis_index("s") * NC + lax.axis_index("c")
    base = pl.multiple_of(wid * BPW, BPW)
    pltpu.sync_copy(x_hbm.at[pl.ds(base * SEQ, BPW * SEQ)], idx_v)

    def per_example(i, _):
        off = pl.multiple_of(i * SEQ, 8)
        cp0 = pltpu.make_async_copy(
            table_hbm.at[idx_v.at[pl.ds(off, C0)]],
            rows_v.at[pl.ds(0, C0)], sem)
        cp1 = pltpu.make_async_copy(
            table_hbm.at[idx_v.at[pl.ds(off + C0, C1)]],
            rows_v.at[pl.ds(C0, C1)], sem)
        cp0.start()
        cp1.start()
        cp0.wait()
        cp1.wait()

        def body(s, accs):
            new = []
            for j in range(2):
                for k in range(EG):
                    new.append(accs[j * EG + k]
                               + rows_v[j * HALF + s, pl.ds(k * LANES, LANES)])
            return tuple(new)

        zero = jnp.zeros((LANES,), jnp.float32)
        accs = lax.fori_loop(0, HALF, body, (zero,) * (2 * EG))
        for k in range(EG):
            out_v[i, pl.ds(k * LANES, LANES)] = accs[k] + accs[EG + k]
        return 0

    lax.fori_loop(0, BPW, per_example, 0)
    pltpu.sync_copy(out_v, out_hbm.at[pl.ds(base, BPW)])


def _pooled_sum(x, table):
    mesh = plsc.VectorSubcoreMesh(core_axis_name="c", subcore_axis_name="s")
    f = functools.partial(
        pl.kernel,
        mesh=mesh,
        out_type=jax.ShapeDtypeStruct((BATCH, EMBED), jnp.float32),
        scratch_types=[
            pltpu.VMEM((BPW * SEQ,), jnp.int32),
            pltpu.VMEM((SEQ, EMBED), jnp.float32),
            pltpu.VMEM((BPW, EMBED), jnp.float32),
            pltpu.SemaphoreType.DMA,
        ],
        compiler_params=pltpu.CompilerParams(use_tc_tiling_on_sc=False),
    )(_pool_body)
    return f(x, table)


def _mlp_body(p_ref, w1_ref, b1_ref, w2_ref, b2_ref, w3_ref, b3_ref, o_ref):
    p = p_ref[...] * (1.0 / SEQ)
    h = jnp.dot(p, w1_ref[...], preferred_element_type=jnp.float32)
    h = jnp.maximum(h + b1_ref[...], 0.0)
    h = jnp.dot(h, w2_ref[...], preferred_element_type=jnp.float32)
    h = jnp.maximum(h + b2_ref[...], 0.0)
    o_ref[...] = jnp.dot(h, w3_ref[...],
                         preferred_element_type=jnp.float32) + b3_ref[...]


def _mlp(pooled_sum, W1, b1, W2, b2, W3, b3):
    pad = 128 - CLASSES
    W3p = jnp.pad(W3, ((0, 0), (0, pad)))
    b3p = jnp.pad(b3, (0, pad)).reshape(1, 128)
    out = pl.pallas_call(
        _mlp_body,
        out_shape=jax.ShapeDtypeStruct((BATCH, 128), jnp.float32),
    )(pooled_sum, W1, b1.reshape(1, HIDDEN), W2, b2.reshape(1, HIDDEN),
      W3p, b3p)
    return out[:, :CLASSES]


def kernel(x, table, W1, b1, W2, b2, W3, b3):
    x = x.astype(jnp.int32).reshape(BATCH * SEQ)
    pooled_sum = _pooled_sum(x, table)
    return _mlp(pooled_sum, W1, b1, W2, b2, W3, b3)
